# baseline (speedup 1.0000x reference)
import functools

import jax
import jax.numpy as jnp
from jax import lax
from jax.experimental import pallas as pl
from jax.experimental.pallas import tpu as pltpu
from jax.experimental.pallas import tpu_sc as plsc

P = 50000
B = 2
C = 64
GX = 512
GY = 512
NCELL = B * GX * GY

NC, NS, L = 2, 16, 16
NWORK = NC * NS
PER_W = 1568
P_PAD = PER_W * NWORK
GRP = PER_W // L

XB = 32


def _zero_mask():
    def body(o_ref):
        o_ref[...] = jnp.zeros_like(o_ref)

    return pl.pallas_call(
        body,
        out_shape=jax.ShapeDtypeStruct((B * GX, GY), jnp.int32),
        grid=(2,),
        out_specs=pl.BlockSpec((B * GX // 2, GY), lambda i: (i, 0)),
    )()


_mesh = plsc.VectorSubcoreMesh(core_axis_name="c", subcore_axis_name="s")


@functools.partial(
    pl.kernel,
    out_type=jax.ShapeDtypeStruct((NCELL, C), jnp.bfloat16),
    mesh=_mesh,
    compiler_params=pltpu.CompilerParams(use_tc_tiling_on_sc=False),
    scratch_types=[
        pltpu.VMEM((PER_W,), jnp.int32),
        pltpu.VMEM((PER_W,), jnp.int32),
        pltpu.VMEM((PER_W,), jnp.int32),
        pltpu.VMEM((PER_W,), jnp.int32),
        pltpu.VMEM((PER_W,), jnp.int32),
        pltpu.VMEM((PER_W, C), jnp.bfloat16),
        pltpu.SemaphoreType.DMA,
        pltpu.SemaphoreType.DMA,
    ],
)
def _sc_scatter(feat_hbm, b_hbm, x_hbm, y_hbm, mask_ref, t_ref,
                b_v, x_v, y_v, q_v, ones_v, feat_v, sem_t, sem_m):
    wid = lax.axis_index("s") * NC + lax.axis_index("c")
    p0 = wid * PER_W
    cp_feat = pltpu.async_copy(feat_hbm.at[pl.ds(p0, PER_W)], feat_v, sem_t)
    pltpu.sync_copy(b_hbm.at[pl.ds(p0, PER_W)], b_v)
    pltpu.sync_copy(x_hbm.at[pl.ds(p0, PER_W)], x_v)
    pltpu.sync_copy(y_hbm.at[pl.ds(p0, PER_W)], y_v)

    def build(g, carry):
        sl = pl.ds(g * L, L)
        q_v[sl] = b_v[sl] * (GX * GY) + x_v[sl] * GY + y_v[sl]
        ones_v[sl] = jnp.ones((L,), jnp.int32)
        return carry

    lax.fori_loop(0, GRP, build, 0)
    cp_feat.wait()
    cp_mask = pltpu.async_copy(ones_v, mask_ref.at[q_v], sem_m)
    pltpu.async_copy(feat_v, t_ref.at[q_v], sem_t).wait()
    cp_mask.wait()


def _transpose_masked(mask2d, t):
    def body(m_ref, t_ref, o_ref):
        tt = jnp.transpose(t_ref[...], (1, 0)).astype(jnp.float32)
        m = m_ref[...].reshape(1, XB, GY)
        o_ref[...] = jnp.where(m != 0, tt.reshape(C, XB, GY), 0.0)[None]

    return pl.pallas_call(
        body,
        grid=(B * GX // XB,),
        in_specs=[
            pl.BlockSpec((XB, GY), lambda g: (g, 0)),
            pl.BlockSpec((XB * GY, C), lambda g: (g, 0)),
        ],
        out_specs=pl.BlockSpec(
            (1, C, XB, GY),
            lambda g: (g // (GX // XB), 0, g % (GX // XB), 0),
        ),
        out_shape=jax.ShapeDtypeStruct((B, C, GX, GY), jnp.float32),
    )(mask2d, t)


def kernel(pillar_features, pillar_coords, batch_size):
    del batch_size
    pad = P_PAD - P
    b = pillar_coords[:, 0]
    x = pillar_coords[:, 1]
    y = pillar_coords[:, 2]
    featp = jnp.concatenate(
        [pillar_features, jnp.broadcast_to(pillar_features[0], (pad, C))], 0
    ).astype(jnp.bfloat16)
    bp = jnp.concatenate([b, jnp.broadcast_to(b[0], (pad,))])
    xp = jnp.concatenate([x, jnp.broadcast_to(x[0], (pad,))])
    yp = jnp.concatenate([y, jnp.broadcast_to(y[0], (pad,))])
    mask_ref = jax.new_ref(_zero_mask().reshape(NCELL))
    t = _sc_scatter(featp, bp, xp, yp, mask_ref)
    mask2d = jax.freeze(mask_ref).reshape(B * GX, GY)
    return _transpose_masked(mask2d, t)


# speedup vs baseline: 1.2590x; 1.2590x over previous
import functools

import jax
import jax.numpy as jnp
from jax import lax
from jax.experimental import pallas as pl
from jax.experimental.pallas import tpu as pltpu
from jax.experimental.pallas import tpu_sc as plsc

P = 50000
B = 2
C = 64
GX = 512
GY = 512
NCELL = B * GX * GY

NC, NS, L = 2, 16, 16
NWORK = NC * NS
PER_W = 1568
P_PAD = PER_W * NWORK
GRP = PER_W // L

XB = 64


def _zero_mask():
    def body(o_ref):
        o_ref[...] = jnp.zeros_like(o_ref)

    return pl.pallas_call(
        body,
        out_shape=jax.ShapeDtypeStruct((B * GX, GY), jnp.int32),
        grid=(2,),
        out_specs=pl.BlockSpec((B * GX // 2, GY), lambda i: (i, 0)),
    )()


_mesh = plsc.VectorSubcoreMesh(core_axis_name="c", subcore_axis_name="s")


@functools.partial(
    pl.kernel,
    out_type=jax.ShapeDtypeStruct((NCELL, C), jnp.float32),
    mesh=_mesh,
    compiler_params=pltpu.CompilerParams(use_tc_tiling_on_sc=False),
    scratch_types=[
        pltpu.VMEM((PER_W,), jnp.int32),
        pltpu.VMEM((PER_W,), jnp.int32),
        pltpu.VMEM((PER_W,), jnp.int32),
        pltpu.VMEM((PER_W,), jnp.int32),
        pltpu.VMEM((PER_W,), jnp.int32),
        pltpu.VMEM((PER_W, C), jnp.float32),
        pltpu.SemaphoreType.DMA,
        pltpu.SemaphoreType.DMA,
    ],
)
def _sc_scatter(feat_hbm, b_hbm, x_hbm, y_hbm, mask_ref, t_ref,
                b_v, x_v, y_v, q_v, ones_v, feat_v, sem_t, sem_m):
    wid = lax.axis_index("s") * NC + lax.axis_index("c")
    p0 = wid * PER_W
    cp_feat = pltpu.async_copy(feat_hbm.at[pl.ds(p0, PER_W)], feat_v, sem_t)
    pltpu.sync_copy(b_hbm.at[pl.ds(p0, PER_W)], b_v)
    pltpu.sync_copy(x_hbm.at[pl.ds(p0, PER_W)], x_v)
    pltpu.sync_copy(y_hbm.at[pl.ds(p0, PER_W)], y_v)

    def build(g, carry):
        sl = pl.ds(g * L, L)
        q_v[sl] = b_v[sl] * (GX * GY) + x_v[sl] * GY + y_v[sl]
        ones_v[sl] = jnp.ones((L,), jnp.int32)
        return carry

    lax.fori_loop(0, GRP, build, 0)
    cp_feat.wait()
    cp_mask = pltpu.async_copy(ones_v, mask_ref.at[q_v], sem_m)
    pltpu.async_copy(feat_v, t_ref.at[q_v], sem_t).wait()
    cp_mask.wait()


def _transpose_masked(mask2d, t):
    def body(m_ref, t_ref, o_ref):
        tt = jnp.transpose(t_ref[...], (1, 0))
        m = m_ref[...].reshape(1, XB, GY)
        o_ref[...] = jnp.where(m != 0, tt.reshape(C, XB, GY), 0.0)[None]

    return pl.pallas_call(
        body,
        grid=(B * GX // XB,),
        in_specs=[
            pl.BlockSpec((XB, GY), lambda g: (g, 0)),
            pl.BlockSpec((XB * GY, C), lambda g: (g, 0)),
        ],
        out_specs=pl.BlockSpec(
            (1, C, XB, GY),
            lambda g: (g // (GX // XB), 0, g % (GX // XB), 0),
        ),
        out_shape=jax.ShapeDtypeStruct((B, C, GX, GY), jnp.float32),
    )(mask2d, t)


def kernel(pillar_features, pillar_coords, batch_size):
    del batch_size
    pad = P_PAD - P
    b = pillar_coords[:, 0]
    x = pillar_coords[:, 1]
    y = pillar_coords[:, 2]
    featp = jnp.concatenate(
        [pillar_features, jnp.broadcast_to(pillar_features[0], (pad, C))], 0
    )
    bp = jnp.concatenate([b, jnp.broadcast_to(b[0], (pad,))])
    xp = jnp.concatenate([x, jnp.broadcast_to(x[0], (pad,))])
    yp = jnp.concatenate([y, jnp.broadcast_to(y[0], (pad,))])
    mask_ref = jax.new_ref(_zero_mask().reshape(NCELL))
    t = _sc_scatter(featp, bp, xp, yp, mask_ref)
    mask2d = jax.freeze(mask_ref).reshape(B * GX, GY)
    return _transpose_masked(mask2d, t)


# speedup vs baseline: 2.3728x; 1.8847x over previous
import functools

import jax
import jax.numpy as jnp
from jax import lax
from jax.experimental import pallas as pl
from jax.experimental.pallas import tpu as pltpu
from jax.experimental.pallas import tpu_sc as plsc

P = 50000
B = 2
C = 64
GX = 512
GY = 512
NCELL = B * GX * GY

NC, NS, L = 2, 16, 16
NWORK = NC * NS
PER_W = 1568
GRP = PER_W // L

XB = 32


def _zero_mask():
    def body(o_ref):
        o_ref[...] = jnp.zeros_like(o_ref)

    return pl.pallas_call(
        body,
        out_shape=jax.ShapeDtypeStruct((B * GX, GY), jnp.int32),
        grid=(2,),
        out_specs=pl.BlockSpec((B * GX // 2, GY), lambda i: (i, 0)),
    )()


_mesh = plsc.VectorSubcoreMesh(core_axis_name="c", subcore_axis_name="s")


@functools.partial(
    pl.kernel,
    out_type=jax.ShapeDtypeStruct((2 * NCELL, C), jnp.float32),
    mesh=_mesh,
    compiler_params=pltpu.CompilerParams(use_tc_tiling_on_sc=False),
    scratch_types=[
        pltpu.VMEM((PER_W,), jnp.int32),
        pltpu.VMEM((PER_W,), jnp.int32),
        pltpu.VMEM((PER_W,), jnp.int32),
        pltpu.VMEM((PER_W,), jnp.int32),
        pltpu.VMEM((PER_W,), jnp.int32),
        pltpu.VMEM((PER_W,), jnp.int32),
        pltpu.VMEM((PER_W, C), jnp.float32),
        pltpu.SemaphoreType.DMA,
        pltpu.SemaphoreType.DMA,
    ],
)
def _sc_scatter(feat_hbm, ct_hbm, mask_ref, t2_ref,
                b_v, x_v, y_v, q_v, q2_v, ones_v, feat_v, sem_t, sem_m):
    wid = lax.axis_index("s") * NC + lax.axis_index("c")
    p0 = jnp.minimum(wid * PER_W, P - PER_W)
    cp_feat = pltpu.async_copy(feat_hbm.at[pl.ds(p0, PER_W)], feat_v, sem_t)
    pltpu.sync_copy(ct_hbm.at[0, pl.ds(p0, PER_W)], b_v)
    pltpu.sync_copy(ct_hbm.at[1, pl.ds(p0, PER_W)], x_v)
    pltpu.sync_copy(ct_hbm.at[2, pl.ds(p0, PER_W)], y_v)

    def build(g, carry):
        sl = pl.ds(g * L, L)
        q = b_v[sl] * (GX * GY) + x_v[sl] * GY + y_v[sl]
        q_v[sl] = q
        q2_v[sl] = q + q
        ones_v[sl] = jnp.ones((L,), jnp.int32)
        return carry

    lax.fori_loop(0, GRP, build, 0)
    cp_feat.wait()
    cp_mask = pltpu.async_copy(ones_v, mask_ref.at[q_v], sem_m)
    pltpu.async_copy(feat_v, t2_ref.at[q2_v], sem_t).wait()
    cp_mask.wait()


def _transpose_masked(mask2d, t128):
    def body(m_ref, t_ref, o_ref):
        tt = jnp.transpose(t_ref[:, :C], (1, 0))
        m = m_ref[...].reshape(1, XB, GY)
        o_ref[...] = jnp.where(m != 0, tt.reshape(C, XB, GY), 0.0)[None]

    return pl.pallas_call(
        body,
        grid=(B * GX // XB,),
        in_specs=[
            pl.BlockSpec((XB, GY), lambda g: (g, 0)),
            pl.BlockSpec((XB * GY, 2 * C), lambda g: (g, 0)),
        ],
        out_specs=pl.BlockSpec(
            (1, C, XB, GY),
            lambda g: (g // (GX // XB), 0, g % (GX // XB), 0),
        ),
        out_shape=jax.ShapeDtypeStruct((B, C, GX, GY), jnp.float32),
    )(mask2d, t128)


def kernel(pillar_features, pillar_coords, batch_size):
    del batch_size
    coords_t = pillar_coords.T
    mask_ref = jax.new_ref(_zero_mask().reshape(NCELL))
    t2 = _sc_scatter(pillar_features, coords_t, mask_ref)
    mask2d = jax.freeze(mask_ref).reshape(B * GX, GY)
    return _transpose_masked(mask2d, t2.reshape(NCELL, 2 * C))


# speedup vs baseline: 2.4302x; 1.0242x over previous
import functools

import jax
import jax.numpy as jnp
from jax import lax
from jax.experimental import pallas as pl
from jax.experimental.pallas import tpu as pltpu
from jax.experimental.pallas import tpu_sc as plsc

P = 50000
B = 2
C = 64
GX = 512
GY = 512
NCELL = B * GX * GY

NC, NS, L = 2, 16, 16
NWORK = NC * NS
PER_W = 1568
GRP = PER_W // L

XB = 64


def _zero_mask():
    def body(o_ref):
        o_ref[...] = jnp.zeros_like(o_ref)

    return pl.pallas_call(
        body,
        out_shape=jax.ShapeDtypeStruct((B * GX, GY), jnp.int32),
        grid=(2,),
        out_specs=pl.BlockSpec((B * GX // 2, GY), lambda i: (i, 0)),
    )()


_mesh = plsc.VectorSubcoreMesh(core_axis_name="c", subcore_axis_name="s")


@functools.partial(
    pl.kernel,
    out_type=jax.ShapeDtypeStruct((2 * NCELL, C), jnp.float32),
    mesh=_mesh,
    compiler_params=pltpu.CompilerParams(use_tc_tiling_on_sc=False),
    scratch_types=[
        pltpu.VMEM((PER_W,), jnp.int32),
        pltpu.VMEM((PER_W,), jnp.int32),
        pltpu.VMEM((PER_W,), jnp.int32),
        pltpu.VMEM((PER_W,), jnp.int32),
        pltpu.VMEM((PER_W,), jnp.int32),
        pltpu.VMEM((PER_W,), jnp.int32),
        pltpu.VMEM((PER_W, C), jnp.float32),
        pltpu.SemaphoreType.DMA,
        pltpu.SemaphoreType.DMA,
    ],
)
def _sc_scatter(feat_hbm, ct_hbm, mask_ref, t2_ref,
                b_v, x_v, y_v, q_v, q2_v, ones_v, feat_v, sem_t, sem_m):
    wid = lax.axis_index("s") * NC + lax.axis_index("c")
    p0 = jnp.minimum(wid * PER_W, P - PER_W)
    cp_feat = pltpu.async_copy(feat_hbm.at[pl.ds(p0, PER_W)], feat_v, sem_t)
    pltpu.sync_copy(ct_hbm.at[0, pl.ds(p0, PER_W)], b_v)
    pltpu.sync_copy(ct_hbm.at[1, pl.ds(p0, PER_W)], x_v)
    pltpu.sync_copy(ct_hbm.at[2, pl.ds(p0, PER_W)], y_v)

    def build(g, carry):
        sl = pl.ds(g * L, L)
        q = b_v[sl] * (GX * GY) + x_v[sl] * GY + y_v[sl]
        q_v[sl] = q
        q2_v[sl] = q + q
        ones_v[sl] = jnp.ones((L,), jnp.int32)
        return carry

    lax.fori_loop(0, GRP, build, 0)
    cp_feat.wait()
    cp_mask = pltpu.async_copy(ones_v, mask_ref.at[q_v], sem_m)
    pltpu.async_copy(feat_v, t2_ref.at[q2_v], sem_t).wait()
    cp_mask.wait()


def _transpose_masked(mask2d, t128):
    def body(m_ref, t_ref, o_ref):
        tt = jnp.transpose(t_ref[:, :C], (1, 0))
        m = m_ref[...].reshape(1, XB, GY)
        o_ref[...] = jnp.where(m != 0, tt.reshape(C, XB, GY), 0.0)[None]

    return pl.pallas_call(
        body,
        grid=(B * GX // XB,),
        in_specs=[
            pl.BlockSpec((XB, GY), lambda g: (g, 0)),
            pl.BlockSpec((XB * GY, 2 * C), lambda g: (g, 0)),
        ],
        out_specs=pl.BlockSpec(
            (1, C, XB, GY),
            lambda g: (g // (GX // XB), 0, g % (GX // XB), 0),
        ),
        out_shape=jax.ShapeDtypeStruct((B, C, GX, GY), jnp.float32),
    )(mask2d, t128)


def kernel(pillar_features, pillar_coords, batch_size):
    del batch_size
    coords_t = pillar_coords.T
    mask_ref = jax.new_ref(_zero_mask().reshape(NCELL))
    t2 = _sc_scatter(pillar_features, coords_t, mask_ref)
    mask2d = jax.freeze(mask_ref).reshape(B * GX, GY)
    return _transpose_masked(mask2d, t2.reshape(NCELL, 2 * C))
